# baseline (device time: 57275 ns/iter reference)
import jax
import jax.numpy as jnp
from jax import lax
from jax.experimental import pallas as pl
from jax.experimental.pallas import tpu as pltpu

N_Y = 4
D_ROWS = 384
S_ROWS = 320
S_BASE = 2 * D_ROWS


def kernel(x):
    m_per, n = x.shape
    assert m_per == 2 * D_ROWS + 4 * S_ROWS

    def body(x_ref, out_ref, ys_s, y_r, xdel_s, xdel_r, xsig_s, xsig_r,
             zdel_s, zdel_r, zsig_s, zsig_r, xrel_s, xrel_r, zrel_s, zrel_r):
        my_x = lax.axis_index("x")
        my_y = lax.axis_index("y")
        my_z = lax.axis_index("z")
        zp = my_z % 2
        cls = (my_x + zp) % 2
        partner = (1 - my_x, my_y, my_z)
        buddy = (my_x, my_y, my_z - 2 * zp + 1)

        def rng(c, off, rows):
            return out_ref.at[pl.ds(c * m_per + off, rows), :]

        def dset(c, cl):
            return rng(c, cl * D_ROWS, D_ROWS)

        def dhalf(c, cl, h):
            return rng(c, cl * D_ROWS + h * (D_ROWS // 2), D_ROWS // 2)

        def soff(xx, zz):
            return S_BASE + S_ROWS * (2 * zz + xx)

        def sset(c, xx, zz):
            return rng(c, soff(xx, zz), S_ROWS)

        def shalf(c, xx, zz, h):
            return rng(c, soff(xx, zz) + h * (S_ROWS // 2), S_ROWS // 2)

        def copy(src, dst, ssem, rsem, dev):
            return pltpu.make_async_remote_copy(
                src_ref=src, dst_ref=dst, send_sem=ssem, recv_sem=rsem,
                device_id=dev, device_id_type=pl.DeviceIdType.MESH,
            )

        def slot(src_y):
            return jnp.where(src_y < my_y, src_y, src_y - 1)

        def sel(table):
            v = jnp.int32(table[3])
            for yy in (2, 1, 0):
                v = jnp.where(my_y == yy, table[yy], v)
            return v

        srcs = [sel(t) for t in ([1, 0, 1, 2], [2, 2, 3, 1], [3, 3, 0, 0])]

        barrier_sem = pltpu.get_barrier_semaphore()
        peers = [(my_x, (my_y + 1 + k) % N_Y, my_z) for k in range(3)]
        peers += [partner, buddy]
        for dev in peers:
            pl.semaphore_signal(
                barrier_sem, inc=1, device_id=dev,
                device_id_type=pl.DeviceIdType.MESH,
            )
        pl.semaphore_wait(barrier_sem, len(peers))

        out_ref[pl.ds(my_y * m_per + cls * D_ROWS, D_ROWS), :] = (
            x_ref[pl.ds(cls * D_ROWS, D_ROWS), :].astype(jnp.bfloat16)
        )
        s_me = soff(my_x, zp)
        out_ref[pl.ds(my_y * m_per + s_me, S_ROWS), :] = (
            x_ref[pl.ds(s_me, S_ROWS), :].astype(jnp.bfloat16)
        )

        for t in range(3):
            y_t = (my_y + 1 + t) % N_Y
            r_slot = jnp.where(my_y < y_t, my_y, my_y - 1)
            tgt = (my_x, y_t, my_z)
            copy(dset(my_y, cls), dset(my_y, cls),
                 ys_s.at[t, 0], y_r.at[r_slot, 0], tgt).start()
            copy(sset(my_y, my_x, zp), sset(my_y, my_x, zp),
                 ys_s.at[t, 1], y_r.at[r_slot, 1], tgt).start()

        out_ref[pl.ds(my_y * m_per + (1 - cls) * D_ROWS, D_ROWS), :] = (
            x_ref[pl.ds((1 - cls) * D_ROWS, D_ROWS), :].astype(jnp.bfloat16)
        )
        for d in range(1, 4):
            so = soff((my_x + d) % 2, (zp + d // 2) % 2)
            out_ref[pl.ds(my_y * m_per + so, S_ROWS), :] = (
                x_ref[pl.ds(so, S_ROWS), :].astype(jnp.bfloat16)
            )

        for j in range(3):
            src = srcs[j]
            s = slot(src)
            copy(dset(src, cls), dset(src, cls),
                 ys_s.at[0, 0], y_r.at[s, 0], partner).wait_recv()
            copy(dhalf(src, cls, zp), dhalf(src, cls, zp),
                 xdel_s.at[s], xdel_r.at[s], partner).start()
            copy(dhalf(src, cls, zp), dhalf(src, cls, zp),
                 zdel_s.at[s], zdel_r.at[s], buddy).start()
            copy(sset(src, my_x, zp), sset(src, my_x, zp),
                 ys_s.at[0, 1], y_r.at[s, 1], partner).wait_recv()
            copy(sset(src, my_x, zp), sset(src, my_x, zp),
                 xsig_s.at[s], xsig_r.at[s], partner).start()
            copy(sset(src, my_x, zp), sset(src, my_x, zp),
                 zsig_s.at[s], zsig_r.at[s], buddy).start()

        for j in range(3):
            src = srcs[j]
            s = slot(src)
            copy(sset(src, 1 - my_x, zp), sset(src, 1 - my_x, zp),
                 xsig_s.at[s], xsig_r.at[s], partner).wait_recv()
            copy(shalf(src, 1 - my_x, zp, my_x), shalf(src, 1 - my_x, zp, my_x),
                 zrel_s.at[s], zrel_r.at[s], buddy).start()
            copy(sset(src, my_x, 1 - zp), sset(src, my_x, 1 - zp),
                 zsig_s.at[s], zsig_r.at[s], buddy).wait_recv()
            copy(shalf(src, my_x, 1 - zp, my_x), shalf(src, my_x, 1 - zp, my_x),
                 xrel_s.at[s], xrel_r.at[s], partner).start()

        for j in range(3):
            src = srcs[j]
            s = slot(src)
            copy(dhalf(src, 1 - cls, zp), dhalf(src, 1 - cls, zp),
                 xdel_s.at[s], xdel_r.at[s], partner).wait_recv()
            copy(dhalf(src, 1 - cls, 1 - zp), dhalf(src, 1 - cls, 1 - zp),
                 zdel_s.at[s], zdel_r.at[s], buddy).wait_recv()
            copy(shalf(src, 1 - my_x, 1 - zp, 1 - my_x),
                 shalf(src, 1 - my_x, 1 - zp, 1 - my_x),
                 xrel_s.at[s], xrel_r.at[s], partner).wait_recv()
            copy(shalf(src, 1 - my_x, 1 - zp, my_x),
                 shalf(src, 1 - my_x, 1 - zp, my_x),
                 zrel_s.at[s], zrel_r.at[s], buddy).wait_recv()

        for t in range(3):
            y_t = (my_y + 1 + t) % N_Y
            tgt = (my_x, y_t, my_z)
            copy(dset(my_y, cls), dset(my_y, cls),
                 ys_s.at[t, 0], y_r.at[0, 0], tgt).wait_send()
            copy(sset(my_y, my_x, zp), sset(my_y, my_x, zp),
                 ys_s.at[t, 1], y_r.at[0, 1], tgt).wait_send()
        for j in range(3):
            src = srcs[j]
            s = slot(src)
            copy(dhalf(src, cls, zp), dhalf(src, cls, zp),
                 xdel_s.at[s], xdel_r.at[s], partner).wait_send()
            copy(dhalf(src, cls, zp), dhalf(src, cls, zp),
                 zdel_s.at[s], zdel_r.at[s], buddy).wait_send()
            copy(sset(src, my_x, zp), sset(src, my_x, zp),
                 xsig_s.at[s], xsig_r.at[s], partner).wait_send()
            copy(sset(src, my_x, zp), sset(src, my_x, zp),
                 zsig_s.at[s], zsig_r.at[s], buddy).wait_send()
            copy(shalf(src, 1 - my_x, zp, my_x), shalf(src, 1 - my_x, zp, my_x),
                 zrel_s.at[s], zrel_r.at[s], buddy).wait_send()
            copy(shalf(src, my_x, 1 - zp, my_x), shalf(src, my_x, 1 - zp, my_x),
                 xrel_s.at[s], xrel_r.at[s], partner).wait_send()

    dma = pltpu.SemaphoreType.DMA
    return pl.pallas_call(
        body,
        out_shape=jax.ShapeDtypeStruct((N_Y * m_per, n), jnp.bfloat16),
        in_specs=[pl.BlockSpec(memory_space=pltpu.VMEM)],
        out_specs=pl.BlockSpec(memory_space=pltpu.VMEM),
        scratch_shapes=[
            dma((3, 2)), dma((3, 2)),
            dma((3,)), dma((3,)),
            dma((3,)), dma((3,)),
            dma((3,)), dma((3,)),
            dma((3,)), dma((3,)),
            dma((3,)), dma((3,)),
            dma((3,)), dma((3,)),
        ],
        compiler_params=pltpu.CompilerParams(collective_id=0),
    )(x)
